# baseline (speedup 1.0000x reference)
import jax
import jax.numpy as jnp
from jax import lax
from jax.experimental import pallas as pl
from jax.experimental.pallas import tpu as pltpu
from jax.experimental.pallas import tpu_sc as plsc

VOCAB = 1000000
EMB = 64
IMG_FEAT = 2048
HIDDEN = 256
N_WORDS = 819200

_NC = 2
_NS = 16
_NW = _NC * _NS
_PER_W = N_WORDS // _NW
_CHUNK = 128
_NCHUNK = _PER_W // _CHUNK
_GROUP = 25
_NGROUP = _NCHUNK // _GROUP

_ZLEN = 25600
_DLEN = 62504


def _stripe_start(s):
    return s * (VOCAB // 16) - 4 * (s % 2)


def _sc_hist(idx_hbm, out0, out1, idx_v, zbuf, ones_v, csh, sem):
    c = lax.axis_index("c")
    s = lax.axis_index("s")
    wid = c * _NS + s

    pltpu.sync_copy(idx_hbm.at[wid], idx_v)

    z16 = jnp.zeros((16,), jnp.float32)
    o16 = jnp.ones((16,), jnp.float32)

    def zfill(i, _):
        zbuf[pl.ds(i * 16, 16)] = z16
        return 0

    lax.fori_loop(0, _ZLEN // 16, zfill, 0, unroll=8)
    for k in range(_CHUNK // 16):
        ones_v[pl.ds(k * 16, 16)] = o16

    start = pl.multiple_of(_stripe_start(s), 8)
    pltpu.sync_copy(zbuf.at[pl.ds(0, _ZLEN)], csh.at[pl.ds(start, _ZLEN)])
    pltpu.sync_copy(zbuf.at[pl.ds(0, _ZLEN)],
                    csh.at[pl.ds(start + _ZLEN, _ZLEN)])
    pltpu.sync_copy(zbuf.at[pl.ds(0, _DLEN - 2 * _ZLEN)],
                    csh.at[pl.ds(start + 2 * _ZLEN, _DLEN - 2 * _ZLEN)])

    plsc.subcore_barrier()

    def group_body(j, _):
        for k in range(_GROUP):
            pltpu.async_copy(
                ones_v, csh.at[idx_v.at[j * _GROUP + k]], sem, add=True)
        pltpu.make_async_copy(
            out0.at[0].at[pl.ds(0, _GROUP * _CHUNK)],
            zbuf.at[pl.ds(0, _GROUP * _CHUNK)],
            sem,
        ).wait()
        return 0

    lax.fori_loop(0, _NGROUP, group_body, 0)

    plsc.subcore_barrier()

    @pl.when(c == 0)
    def _():
        pltpu.sync_copy(csh.at[pl.ds(start, _DLEN)],
                        out0.at[0].at[pl.ds(start, _DLEN)])

    @pl.when(c == 1)
    def _():
        pltpu.sync_copy(csh.at[pl.ds(start, _DLEN)],
                        out1.at[0].at[pl.ds(start, _DLEN)])


_sc_hist_call = pl.kernel(
    _sc_hist,
    out_type=(
        jax.ShapeDtypeStruct((1, VOCAB), jnp.float32),
        jax.ShapeDtypeStruct((1, VOCAB), jnp.float32),
    ),
    scratch_types=[
        pltpu.VMEM((_NCHUNK, _CHUNK), jnp.int32),
        pltpu.VMEM((_ZLEN,), jnp.float32),
        pltpu.VMEM((_CHUNK,), jnp.float32),
        pltpu.VMEM_SHARED((VOCAB,), jnp.float32),
        pltpu.SemaphoreType.DMA,
    ],
    mesh=plsc.VectorSubcoreMesh(core_axis_name="c", subcore_axis_name="s"),
    compiler_params=pltpu.CompilerParams(use_tc_tiling_on_sc=False),
)


_BK = 32768
_NBLK = (VOCAB + _BK - 1) // _BK


def _weighted_sum_mlp(tab_ref, c0_ref, c1_ref, img_ref, wh_ref, bh_ref,
                      wo_ref, bo_ref, out_ref, acc_ref):
    k = pl.program_id(0)

    @pl.when(k == 0)
    def _():
        acc_ref[...] = jnp.zeros_like(acc_ref)

    cnt = c0_ref[...] + c1_ref[...]
    i = lax.broadcasted_iota(jnp.int32, (1, _BK), 1)
    cnt = jnp.where(k * _BK + i < VOCAB, cnt, 0.0)
    acc_ref[...] += tab_ref[...] * cnt

    @pl.when(k == _NBLK - 1)
    def _():
        bow = jnp.sum(acc_ref[...], axis=1, keepdims=True)
        bow = bow.astype(jnp.bfloat16).astype(jnp.float32)
        img = img_ref[...].astype(jnp.bfloat16).astype(jnp.float32)
        h = (
            jnp.sum(bow * wh_ref[:EMB, :], axis=0, keepdims=True)
            + jnp.dot(img, wh_ref[EMB:, :],
                      preferred_element_type=jnp.float32)
            + bh_ref[...]
        )
        o = (jnp.dot(h, wo_ref[...], preferred_element_type=jnp.float32)
             + bo_ref[...])
        out_ref[...] = jax.nn.sigmoid(o)


_weighted_sum_mlp_call = pl.pallas_call(
    _weighted_sum_mlp,
    grid=(_NBLK,),
    in_specs=[
        pl.BlockSpec((EMB, _BK), lambda k: (0, k)),
        pl.BlockSpec((1, _BK), lambda k: (0, k)),
        pl.BlockSpec((1, _BK), lambda k: (0, k)),
        pl.BlockSpec((1, IMG_FEAT), lambda k: (0, 0)),
        pl.BlockSpec((EMB + IMG_FEAT, HIDDEN), lambda k: (0, 0)),
        pl.BlockSpec((1, HIDDEN), lambda k: (0, 0)),
        pl.BlockSpec((HIDDEN, 1), lambda k: (0, 0)),
        pl.BlockSpec((1, 1), lambda k: (0, 0)),
    ],
    out_specs=pl.BlockSpec((1, 1), lambda k: (0, 0)),
    out_shape=jax.ShapeDtypeStruct((1, 1), jnp.float32),
    scratch_shapes=[pltpu.VMEM((EMB, _BK), jnp.float32)],
)


def kernel(word_inputs, image_inputs, emb_table, W_h, b_h, W_o, b_o):
    idx3 = word_inputs.astype(jnp.int32).reshape(_NW, _NCHUNK, _CHUNK)
    counts0, counts1 = _sc_hist_call(idx3)
    prob = _weighted_sum_mlp_call(
        emb_table.T,
        counts0,
        counts1,
        image_inputs.reshape(1, IMG_FEAT),
        W_h,
        b_h.reshape(1, HIDDEN),
        W_o,
        b_o.reshape(1, 1),
    )
    return prob.reshape((1,))


# speedup vs baseline: 1.1997x; 1.1997x over previous
import jax
import jax.numpy as jnp
from jax import lax
from jax.experimental import pallas as pl
from jax.experimental.pallas import tpu as pltpu
from jax.experimental.pallas import tpu_sc as plsc

VOCAB = 1000000
EMB = 64
IMG_FEAT = 2048
HIDDEN = 256
N_WORDS = 819200

_NC = 2
_NS = 16
_NW = _NC * _NS
_PER_W = N_WORDS // _NW
_CHUNK = 128
_NCHUNK = _PER_W // _CHUNK
_GROUP = 25
_NGROUP = _NCHUNK // _GROUP

_ZLEN = 25600
_DLEN = 62504


def _stripe_start(s):
    return s * (VOCAB // 16) - 4 * (s % 2)


def _sc_hist(idx_hbm, out0, idx_v, zbuf, ones_v, csh, sem):
    c = lax.axis_index("c")
    s = lax.axis_index("s")
    wid = c * _NS + s

    pltpu.sync_copy(idx_hbm.at[wid], idx_v)

    z16 = jnp.zeros((16,), jnp.float32)
    o16 = jnp.ones((16,), jnp.float32)

    def zfill(i, _):
        zbuf[pl.ds(i * 16, 16)] = z16
        return 0

    lax.fori_loop(0, _ZLEN // 16, zfill, 0, unroll=8)
    for k in range(_CHUNK // 16):
        ones_v[pl.ds(k * 16, 16)] = o16

    start = pl.multiple_of(_stripe_start(s), 8)
    pltpu.sync_copy(zbuf.at[pl.ds(0, _ZLEN)], csh.at[pl.ds(start, _ZLEN)])
    pltpu.sync_copy(zbuf.at[pl.ds(0, _ZLEN)],
                    csh.at[pl.ds(start + _ZLEN, _ZLEN)])
    pltpu.sync_copy(zbuf.at[pl.ds(0, _DLEN - 2 * _ZLEN)],
                    csh.at[pl.ds(start + 2 * _ZLEN, _DLEN - 2 * _ZLEN)])

    plsc.subcore_barrier()

    def group_body(j, _):
        for k in range(_GROUP):
            pltpu.async_copy(
                ones_v, csh.at[idx_v.at[j * _GROUP + k]], sem, add=True)
        pltpu.make_async_copy(
            out0.at[0].at[pl.ds(0, _GROUP * _CHUNK)],
            zbuf.at[pl.ds(0, _GROUP * _CHUNK)],
            sem,
        ).wait()
        return 0

    lax.fori_loop(0, _NGROUP, group_body, 0)

    plsc.subcore_barrier()

    pltpu.sync_copy(csh.at[pl.ds(start, _DLEN)],
                    out0.at[c].at[pl.ds(start, _DLEN)])


_sc_hist_call = pl.kernel(
    _sc_hist,
    out_type=jax.ShapeDtypeStruct((_NC, VOCAB), jnp.float32),
    scratch_types=[
        pltpu.VMEM((_NCHUNK, _CHUNK), jnp.int32),
        pltpu.VMEM((_ZLEN,), jnp.float32),
        pltpu.VMEM((_CHUNK,), jnp.float32),
        pltpu.VMEM_SHARED((VOCAB,), jnp.float32),
        pltpu.SemaphoreType.DMA,
    ],
    mesh=plsc.VectorSubcoreMesh(core_axis_name="c", subcore_axis_name="s"),
    compiler_params=pltpu.CompilerParams(use_tc_tiling_on_sc=False),
)


_BK = 32768
_NBLK = (VOCAB + _BK - 1) // _BK


def _weighted_sum_mlp(tab_ref, cnt_ref, img_ref, wh_ref, bh_ref,
                      wo_ref, bo_ref, out_ref, acc_ref):
    k = pl.program_id(0)

    @pl.when(k == 0)
    def _():
        acc_ref[...] = jnp.zeros_like(acc_ref)

    cnt = cnt_ref[0:1, :] + cnt_ref[1:2, :]
    i = lax.broadcasted_iota(jnp.int32, (1, _BK), 1)
    cnt = jnp.where(k * _BK + i < VOCAB, cnt, 0.0)
    acc_ref[...] += tab_ref[...] * cnt

    @pl.when(k == _NBLK - 1)
    def _():
        bow = jnp.sum(acc_ref[...], axis=1, keepdims=True)
        bow = bow.astype(jnp.bfloat16).astype(jnp.float32)
        img = img_ref[...].astype(jnp.bfloat16).astype(jnp.float32)
        h = (
            jnp.sum(bow * wh_ref[:EMB, :], axis=0, keepdims=True)
            + jnp.dot(img, wh_ref[EMB:, :],
                      preferred_element_type=jnp.float32)
            + bh_ref[...]
        )
        o = (jnp.dot(h, wo_ref[...], preferred_element_type=jnp.float32)
             + bo_ref[...])
        out_ref[...] = jax.nn.sigmoid(o)


_weighted_sum_mlp_call = pl.pallas_call(
    _weighted_sum_mlp,
    grid=(_NBLK,),
    in_specs=[
        pl.BlockSpec((EMB, _BK), lambda k: (0, k)),
        pl.BlockSpec((_NC, _BK), lambda k: (0, k)),
        pl.BlockSpec((1, IMG_FEAT), lambda k: (0, 0)),
        pl.BlockSpec((EMB + IMG_FEAT, HIDDEN), lambda k: (0, 0)),
        pl.BlockSpec((1, HIDDEN), lambda k: (0, 0)),
        pl.BlockSpec((HIDDEN, 1), lambda k: (0, 0)),
        pl.BlockSpec((1, 1), lambda k: (0, 0)),
    ],
    out_specs=pl.BlockSpec((1, 1), lambda k: (0, 0)),
    out_shape=jax.ShapeDtypeStruct((1, 1), jnp.float32),
    scratch_shapes=[pltpu.VMEM((EMB, _BK), jnp.float32)],
)


def kernel(word_inputs, image_inputs, emb_table, W_h, b_h, W_o, b_o):
    idx3 = word_inputs.astype(jnp.int32).reshape(_NW, _NCHUNK, _CHUNK)
    counts = _sc_hist_call(idx3)
    prob = _weighted_sum_mlp_call(
        emb_table.T,
        counts,
        image_inputs.reshape(1, IMG_FEAT),
        W_h,
        b_h.reshape(1, HIDDEN),
        W_o,
        b_o.reshape(1, 1),
    )
    return prob.reshape((1,))


# speedup vs baseline: 1.2191x; 1.0162x over previous
import jax
import jax.numpy as jnp
from jax import lax
from jax.experimental import pallas as pl
from jax.experimental.pallas import tpu as pltpu
from jax.experimental.pallas import tpu_sc as plsc

VOCAB = 1000000
EMB = 64
IMG_FEAT = 2048
HIDDEN = 256
N_WORDS = 819200

_NC = 2
_NS = 16
_NW = _NC * _NS
_PER_W = N_WORDS // _NW
_CHUNK = 128
_NCHUNK = _PER_W // _CHUNK
_GROUP = 25
_NGROUP = _NCHUNK // _GROUP

_ZLEN = 25600
_DLEN = 62504


def _stripe_start(s):
    return s * (VOCAB // 16) - 4 * (s % 2)


def _sc_hist(idx_hbm, out0, idx_v, zbuf, ones_v, csh, sem):
    c = lax.axis_index("c")
    s = lax.axis_index("s")
    wid = c * _NS + s

    pltpu.sync_copy(idx_hbm.at[wid], idx_v)

    z16 = jnp.zeros((16,), jnp.float32)
    o16 = jnp.ones((16,), jnp.float32)

    def zfill(i, _):
        zbuf[pl.ds(i * 16, 16)] = z16
        return 0

    lax.fori_loop(0, _ZLEN // 16, zfill, 0, unroll=8)
    for k in range(_CHUNK // 16):
        ones_v[pl.ds(k * 16, 16)] = o16

    start = pl.multiple_of(_stripe_start(s), 8)
    pltpu.sync_copy(zbuf.at[pl.ds(0, _ZLEN)], csh.at[pl.ds(start, _ZLEN)])
    pltpu.sync_copy(zbuf.at[pl.ds(0, _ZLEN)],
                    csh.at[pl.ds(start + _ZLEN, _ZLEN)])
    pltpu.sync_copy(zbuf.at[pl.ds(0, _DLEN - 2 * _ZLEN)],
                    csh.at[pl.ds(start + 2 * _ZLEN, _DLEN - 2 * _ZLEN)])

    plsc.subcore_barrier()

    def group_body(j, _):
        for k in range(_GROUP):
            pltpu.async_copy(
                ones_v, csh.at[idx_v.at[j * _GROUP + k]], sem, add=True)
        pltpu.make_async_copy(
            out0.at[0].at[pl.ds(0, _GROUP * _CHUNK)],
            zbuf.at[pl.ds(0, _GROUP * _CHUNK)],
            sem,
        ).wait()
        return 0

    lax.fori_loop(0, _NGROUP, group_body, 0)

    plsc.subcore_barrier()

    pltpu.sync_copy(csh.at[pl.ds(start, _DLEN)],
                    out0.at[c].at[pl.ds(start, _DLEN)])


_sc_hist_call = pl.kernel(
    _sc_hist,
    out_type=jax.ShapeDtypeStruct((_NC, VOCAB), jnp.float32),
    scratch_types=[
        pltpu.VMEM((_NCHUNK, _CHUNK), jnp.int32),
        pltpu.VMEM((_ZLEN,), jnp.float32),
        pltpu.VMEM((_CHUNK,), jnp.float32),
        pltpu.VMEM_SHARED((VOCAB,), jnp.float32),
        pltpu.SemaphoreType.DMA,
    ],
    mesh=plsc.VectorSubcoreMesh(core_axis_name="c", subcore_axis_name="s"),
    compiler_params=pltpu.CompilerParams(use_tc_tiling_on_sc=False),
)


_BK = 32768
_NBLK = (VOCAB + _BK - 1) // _BK


def _weighted_sum_mlp(tab_ref, cnt_ref, img_ref, wh_ref, bh_ref,
                      wo_ref, bo_ref, out_ref, acc_ref):
    k = pl.program_id(0)

    @pl.when(k == 0)
    def _():
        acc_ref[...] = jnp.zeros_like(acc_ref)

    cnt = cnt_ref[0:1, :] + cnt_ref[1:2, :]
    i = lax.broadcasted_iota(jnp.int32, (1, _BK), 1)
    cnt = jnp.where(k * _BK + i < VOCAB, cnt, 0.0)
    acc_ref[...] += jax.lax.dot_general(
        tab_ref[...], cnt, (((1,), (1,)), ((), ())),
        preferred_element_type=jnp.float32)

    @pl.when(k == _NBLK - 1)
    def _():
        bow = acc_ref[...]
        bow = bow.astype(jnp.bfloat16).astype(jnp.float32)
        img = img_ref[...].astype(jnp.bfloat16).astype(jnp.float32)
        h = (
            jnp.sum(bow * wh_ref[:EMB, :], axis=0, keepdims=True)
            + jnp.dot(img, wh_ref[EMB:, :],
                      preferred_element_type=jnp.float32)
            + bh_ref[...]
        )
        o = (jnp.dot(h, wo_ref[...], preferred_element_type=jnp.float32)
             + bo_ref[...])
        out_ref[...] = jax.nn.sigmoid(o)


_weighted_sum_mlp_call = pl.pallas_call(
    _weighted_sum_mlp,
    grid=(_NBLK,),
    in_specs=[
        pl.BlockSpec((EMB, _BK), lambda k: (0, k)),
        pl.BlockSpec((_NC, _BK), lambda k: (0, k)),
        pl.BlockSpec((1, IMG_FEAT), lambda k: (0, 0)),
        pl.BlockSpec((EMB + IMG_FEAT, HIDDEN), lambda k: (0, 0)),
        pl.BlockSpec((1, HIDDEN), lambda k: (0, 0)),
        pl.BlockSpec((HIDDEN, 1), lambda k: (0, 0)),
        pl.BlockSpec((1, 1), lambda k: (0, 0)),
    ],
    out_specs=pl.BlockSpec((1, 1), lambda k: (0, 0)),
    out_shape=jax.ShapeDtypeStruct((1, 1), jnp.float32),
    scratch_shapes=[pltpu.VMEM((EMB, 1), jnp.float32)],
)


def kernel(word_inputs, image_inputs, emb_table, W_h, b_h, W_o, b_o):
    idx3 = word_inputs.astype(jnp.int32).reshape(_NW, _NCHUNK, _CHUNK)
    counts = _sc_hist_call(idx3)
    prob = _weighted_sum_mlp_call(
        emb_table.T,
        counts,
        image_inputs.reshape(1, IMG_FEAT),
        W_h,
        b_h.reshape(1, HIDDEN),
        W_o,
        b_o.reshape(1, 1),
    )
    return prob.reshape((1,))


# speedup vs baseline: 1.2229x; 1.0031x over previous
import jax
import jax.numpy as jnp
from jax import lax
from jax.experimental import pallas as pl
from jax.experimental.pallas import tpu as pltpu
from jax.experimental.pallas import tpu_sc as plsc

VOCAB = 1000000
EMB = 64
IMG_FEAT = 2048
HIDDEN = 256
N_WORDS = 819200

_NC = 2
_NS = 16
_NW = _NC * _NS
_PER_W = N_WORDS // _NW
_CHUNK = 128
_NCHUNK = _PER_W // _CHUNK
_GROUP = 25
_NGROUP = _NCHUNK // _GROUP

_ZLEN = 25600
_DLEN = 62504


def _stripe_start(s):
    return s * (VOCAB // 16) - 4 * (s % 2)


def _sc_hist(idx_hbm, out0, idx_v, zbuf, ones_v, csh, sem):
    c = lax.axis_index("c")
    s = lax.axis_index("s")
    wid = c * _NS + s

    pltpu.sync_copy(idx_hbm.at[wid], idx_v)

    z16 = jnp.zeros((16,), jnp.float32)
    o16 = jnp.ones((16,), jnp.float32)

    def zfill(i, _):
        zbuf[pl.ds(i * 16, 16)] = z16
        return 0

    lax.fori_loop(0, _ZLEN // 16, zfill, 0, unroll=8)
    for k in range(_CHUNK // 16):
        ones_v[pl.ds(k * 16, 16)] = o16

    start = pl.multiple_of(_stripe_start(s), 8)
    pltpu.sync_copy(zbuf.at[pl.ds(0, _ZLEN)], csh.at[pl.ds(start, _ZLEN)])
    pltpu.sync_copy(zbuf.at[pl.ds(0, _ZLEN)],
                    csh.at[pl.ds(start + _ZLEN, _ZLEN)])
    pltpu.sync_copy(zbuf.at[pl.ds(0, _DLEN - 2 * _ZLEN)],
                    csh.at[pl.ds(start + 2 * _ZLEN, _DLEN - 2 * _ZLEN)])

    plsc.subcore_barrier()

    def group_body(j, _):
        for k in range(_GROUP):
            pltpu.async_copy(
                ones_v, csh.at[idx_v.at[j * _GROUP + k]], sem, add=True)
        pltpu.make_async_copy(
            out0.at[0].at[pl.ds(0, _GROUP * _CHUNK)],
            zbuf.at[pl.ds(0, _GROUP * _CHUNK)],
            sem,
        ).wait()
        return 0

    lax.fori_loop(0, _NGROUP, group_body, 0)

    plsc.subcore_barrier()

    pltpu.sync_copy(csh.at[pl.ds(start, _DLEN)],
                    out0.at[c].at[pl.ds(start, _DLEN)])


_sc_hist_call = pl.kernel(
    _sc_hist,
    out_type=jax.ShapeDtypeStruct((_NC, VOCAB), jnp.float32),
    scratch_types=[
        pltpu.VMEM((_NCHUNK, _CHUNK), jnp.int32),
        pltpu.VMEM((_ZLEN,), jnp.float32),
        pltpu.VMEM((_CHUNK,), jnp.float32),
        pltpu.VMEM_SHARED((VOCAB,), jnp.float32),
        pltpu.SemaphoreType.DMA,
    ],
    mesh=plsc.VectorSubcoreMesh(core_axis_name="c", subcore_axis_name="s"),
    compiler_params=pltpu.CompilerParams(use_tc_tiling_on_sc=False),
)


_BK = 32768
_NBLK = (VOCAB + _BK - 1) // _BK


def _weighted_sum_mlp(tab_ref, cnt_ref, img_ref, wh_ref, bh_ref,
                      wo_ref, bo_ref, out_ref, acc_ref):
    k = pl.program_id(0)

    @pl.when(k == 0)
    def _():
        acc_ref[...] = jnp.zeros_like(acc_ref)

    cnt = cnt_ref[0:1, :] + cnt_ref[1:2, :]
    i = lax.broadcasted_iota(jnp.int32, (1, _BK), 1)
    cnt = jnp.where(k * _BK + i < VOCAB, cnt, 0.0)
    acc_ref[...] += jax.lax.dot_general(
        tab_ref[...], cnt, (((1,), (1,)), ((), ())),
        precision=jax.lax.Precision.HIGHEST,
        preferred_element_type=jnp.float32)

    @pl.when(k == _NBLK - 1)
    def _():
        bow = acc_ref[...]
        bow = bow.astype(jnp.bfloat16).astype(jnp.float32)
        img = img_ref[...].astype(jnp.bfloat16).astype(jnp.float32)
        h = (
            jnp.sum(bow * wh_ref[:EMB, :], axis=0, keepdims=True)
            + jnp.dot(img, wh_ref[EMB:, :],
                      preferred_element_type=jnp.float32)
            + bh_ref[...]
        )
        o = (jnp.dot(h, wo_ref[...], preferred_element_type=jnp.float32)
             + bo_ref[...])
        out_ref[...] = jax.nn.sigmoid(o)


_weighted_sum_mlp_call = pl.pallas_call(
    _weighted_sum_mlp,
    grid=(_NBLK,),
    in_specs=[
        pl.BlockSpec((EMB, _BK), lambda k: (0, k)),
        pl.BlockSpec((_NC, _BK), lambda k: (0, k)),
        pl.BlockSpec((1, IMG_FEAT), lambda k: (0, 0)),
        pl.BlockSpec((EMB + IMG_FEAT, HIDDEN), lambda k: (0, 0)),
        pl.BlockSpec((1, HIDDEN), lambda k: (0, 0)),
        pl.BlockSpec((HIDDEN, 1), lambda k: (0, 0)),
        pl.BlockSpec((1, 1), lambda k: (0, 0)),
    ],
    out_specs=pl.BlockSpec((1, 1), lambda k: (0, 0)),
    out_shape=jax.ShapeDtypeStruct((1, 1), jnp.float32),
    scratch_shapes=[pltpu.VMEM((EMB, 1), jnp.float32)],
)


def kernel(word_inputs, image_inputs, emb_table, W_h, b_h, W_o, b_o):
    idx3 = word_inputs.astype(jnp.int32).reshape(_NW, _NCHUNK, _CHUNK)
    counts = _sc_hist_call(idx3)
    prob = _weighted_sum_mlp_call(
        emb_table.T,
        counts,
        image_inputs.reshape(1, IMG_FEAT),
        W_h,
        b_h.reshape(1, HIDDEN),
        W_o,
        b_o.reshape(1, 1),
    )
    return prob.reshape((1,))
